# baseline (speedup 1.0000x reference)
import functools

import jax
import jax.numpy as jnp
from jax import lax
from jax.experimental import pallas as pl
from jax.experimental.pallas import tpu as pltpu
from jax.experimental.pallas import tpu_sc as plsc

IN_C = 128
SPIRAL = 32
OUT_C = 128
N_PTS = 10000

_info = plsc.get_sparse_core_info()
NC = _info.num_cores
NS = _info.num_subcores
L = _info.num_lanes
NW = NC * NS

PC = 16
RPC = PC * SPIRAL
NSTREAM = 4
FOLD = SPIRAL // NSTREAM
NCHUNK = N_PTS // PC
NIT = (NCHUNK + NW - 1) // NW
VPR = OUT_C // L


def _mm_body(x_ref, w_ref, y_ref):
    y_ref[0] = jnp.dot(x_ref[...], w_ref[0],
                       preferred_element_type=jnp.float32)


def _matmul(x2d, wmat3):
    return pl.pallas_call(
        _mm_body,
        grid=(SPIRAL,),
        in_specs=[
            pl.BlockSpec((N_PTS, IN_C), lambda s: (0, 0)),
            pl.BlockSpec((1, IN_C, OUT_C), lambda s: (s, 0, 0)),
        ],
        out_specs=pl.BlockSpec((1, N_PTS, OUT_C), lambda s: (s, 0, 0)),
        out_shape=jax.ShapeDtypeStruct((SPIRAL, N_PTS, OUT_C), jnp.float32),
    )(x2d, wmat3)


_mesh = plsc.VectorSubcoreMesh(core_axis_name="c", subcore_axis_name="s")


@functools.partial(
    pl.kernel,
    out_type=jax.ShapeDtypeStruct((N_PTS, OUT_C), jnp.float32),
    mesh=_mesh,
    scratch_types=[
        pltpu.VMEM((RPC,), jnp.int32),
        pltpu.VMEM((RPC,), jnp.int32),
        pltpu.VMEM((NSTREAM, FOLD * PC), jnp.int32),
        pltpu.VMEM((NSTREAM, FOLD * PC), jnp.int32),
        pltpu.VMEM((FOLD * PC, OUT_C), jnp.float32),
        pltpu.VMEM((FOLD * PC, OUT_C), jnp.float32),
        pltpu.VMEM((PC, OUT_C), jnp.float32),
        pltpu.VMEM((PC, OUT_C), jnp.float32),
        pltpu.VMEM((OUT_C,), jnp.float32),
        pltpu.SemaphoreType.DMA,
        pltpu.SemaphoreType.DMA,
        pltpu.SemaphoreType.DMA,
        pltpu.SemaphoreType.DMA,
        pltpu.SemaphoreType.DMA,
        pltpu.SemaphoreType.DMA,
    ],
)
def _sc_gather(y_hbm, adj_hbm, b_hbm, out_hbm,
               adj0, adj1, idx0, idx1, dst0, dst1, ob0, ob1, bias_v,
               sema0, sema1, semr0, semr1, semo0, semo1):
    wid = lax.axis_index("s") * NC + lax.axis_index("c")
    pltpu.sync_copy(b_hbm, bias_v)

    def chunk_of(i):
        return jnp.minimum(wid + i * NW, NCHUNK - 1)

    def adj_cp(i, adj_v, sema):
        c = chunk_of(i)
        return pltpu.make_async_copy(
            adj_hbm.at[pl.ds(c * RPC, RPC)], adj_v, sema)

    def out_cp(i, ob_v, semo):
        c = chunk_of(i)
        return pltpu.make_async_copy(
            ob_v, out_hbm.at[pl.ds(c * PC, PC)], semo)

    def stage(adj_v, idx_v, dst_v, semr):
        zero = jnp.zeros((L,), jnp.float32)

        def init_body(r, carry):
            for v in range(VPR):
                dst_v[r, pl.ds(v * L, L)] = zero
            return carry

        lax.fori_loop(0, FOLD * PC, init_body, 0)

        for k in range(NSTREAM):
            for g in range(FOLD):
                s = FOLD * k + g
                av = adj_v[pl.ds((s * PC) , PC)]
                idx_v[k, pl.ds(g * PC, PC)] = av + s * N_PTS
            pltpu.async_copy(y_hbm.at[idx_v.at[k]], dst_v, semr, add=True)

    def drain(idx_v, dst_v, semr):
        for k in range(NSTREAM):
            pltpu.make_async_copy(y_hbm.at[idx_v.at[0]], dst_v, semr).wait()

    def fold_elu_zero(i, dst_v, ob_v):
        c = chunk_of(i)

        def f_body(p, carry):
            for v in range(VPR):
                z = bias_v[pl.ds(v * L, L)]
                for g in range(FOLD):
                    z = z + dst_v[g * PC + p, pl.ds(v * L, L)]
                ob_v[p, pl.ds(v * L, L)] = jnp.where(
                    z > 0.0, z, jnp.exp(jnp.minimum(z, 0.0)) - 1.0)
            return carry

        lax.fori_loop(0, PC, f_body, 0)

        @pl.when(c == NCHUNK - 1)
        def _():
            zero = jnp.zeros((L,), jnp.float32)
            for v in range(VPR):
                ob_v[PC - 1, pl.ds(v * L, L)] = zero

    def valid(i):
        return wid + i * NW < NCHUNK

    def step(i, cur, nxt):
        (c_adj, c_idx, c_dst, c_ob, c_sema, c_semr, c_semo) = cur
        (n_adj, n_idx, n_dst, n_ob, n_sema, n_semr, n_semo) = nxt

        @pl.when(i + 1 < NIT)
        def _():
            adj_cp(i + 1, n_adj, n_sema).wait()
            stage(n_adj, n_idx, n_dst, n_semr)

        @pl.when(i + 2 < NIT)
        def _():
            adj_cp(i + 2, c_adj, c_sema).start()

        drain(c_idx, c_dst, c_semr)

        @pl.when((i >= 2) & valid(i - 2))
        def _():
            out_cp(i - 2, c_ob, c_semo).wait()

        fold_elu_zero(i, c_dst, c_ob)

        @pl.when(valid(i))
        def _():
            out_cp(i, c_ob, c_semo).start()

    slot0 = (adj0, idx0, dst0, ob0, sema0, semr0, semo0)
    slot1 = (adj1, idx1, dst1, ob1, sema1, semr1, semo1)

    adj_cp(0, adj0, sema0).start()
    adj_cp(0, adj0, sema0).wait()
    stage(adj0, idx0, dst0, semr0)
    adj_cp(1, adj1, sema1).start()

    def pair_body(g, carry):
        step(2 * g, slot0, slot1)
        step(2 * g + 1, slot1, slot0)
        return carry

    lax.fori_loop(0, NIT // 2, pair_body, 0)

    @pl.when(valid(NIT - 2))
    def _():
        out_cp(NIT - 2, ob0 if (NIT - 2) % 2 == 0 else ob1,
               semo0 if (NIT - 2) % 2 == 0 else semo1).wait()

    @pl.when(valid(NIT - 1))
    def _():
        out_cp(NIT - 1, ob0 if (NIT - 1) % 2 == 0 else ob1,
               semo0 if (NIT - 1) % 2 == 0 else semo1).wait()


def kernel(x, spiral_adj, W, b):
    x2d = x.reshape(N_PTS, IN_C).astype(jnp.bfloat16)
    adj = (spiral_adj.reshape(NCHUNK, PC, NSTREAM, FOLD).astype(jnp.int32)
           .transpose(0, 2, 3, 1).reshape(N_PTS * SPIRAL))
    wmat3 = (W.reshape(OUT_C, SPIRAL, IN_C).transpose(1, 2, 0)
             .astype(jnp.bfloat16))
    y = _matmul(x2d, wmat3)
    yr = y.reshape(SPIRAL * N_PTS, OUT_C)
    out2d = _sc_gather(yr, adj, b)
    return out2d.reshape(1, N_PTS, OUT_C)


# speedup vs baseline: 1.1388x; 1.1388x over previous
import functools

import jax
import jax.numpy as jnp
from jax import lax
from jax.experimental import pallas as pl
from jax.experimental.pallas import tpu as pltpu
from jax.experimental.pallas import tpu_sc as plsc

IN_C = 128
SPIRAL = 32
OUT_C = 128
N_PTS = 10000

_info = plsc.get_sparse_core_info()
NC = _info.num_cores
NS = _info.num_subcores
L = _info.num_lanes
NW = NC * NS

PC = 16
RPC = PC * SPIRAL
NCHUNK = N_PTS // PC
NIT = (NCHUNK + NW - 1) // NW
VPR = OUT_C // L


def _mm_body(x_ref, w_ref, y_ref):
    y_ref[0] = jnp.dot(x_ref[...], w_ref[0],
                       preferred_element_type=jnp.float32)


def _matmul(x2d, wmat3):
    return pl.pallas_call(
        _mm_body,
        grid=(SPIRAL,),
        in_specs=[
            pl.BlockSpec((N_PTS, IN_C), lambda s: (0, 0)),
            pl.BlockSpec((1, IN_C, OUT_C), lambda s: (s, 0, 0)),
        ],
        out_specs=pl.BlockSpec((1, N_PTS, OUT_C), lambda s: (s, 0, 0)),
        out_shape=jax.ShapeDtypeStruct((SPIRAL, N_PTS, OUT_C), jnp.float32),
    )(x2d, wmat3)


_mesh = plsc.VectorSubcoreMesh(core_axis_name="c", subcore_axis_name="s")


@functools.partial(
    pl.kernel,
    out_type=jax.ShapeDtypeStruct((N_PTS, OUT_C), jnp.float32),
    mesh=_mesh,
    scratch_types=[
        pltpu.VMEM((RPC,), jnp.int32),
        pltpu.VMEM((RPC,), jnp.int32),
        pltpu.VMEM((SPIRAL, PC), jnp.int32),
        pltpu.VMEM((SPIRAL, PC), jnp.int32),
        pltpu.VMEM((PC, OUT_C), jnp.float32),
        pltpu.VMEM((PC, OUT_C), jnp.float32),
        pltpu.VMEM((OUT_C,), jnp.float32),
        pltpu.SemaphoreType.DMA,
        pltpu.SemaphoreType.DMA,
        pltpu.SemaphoreType.DMA,
        pltpu.SemaphoreType.DMA,
        pltpu.SemaphoreType.DMA,
        pltpu.SemaphoreType.DMA,
    ],
)
def _sc_gather(y_hbm, adj_hbm, b_hbm, out_hbm,
               adj0, adj1, idx0, idx1, acc0, acc1, bias_v,
               sema0, sema1, semr0, semr1, semo0, semo1):
    wid = lax.axis_index("s") * NC + lax.axis_index("c")
    pltpu.sync_copy(b_hbm, bias_v)

    def chunk_of(i):
        return jnp.minimum(wid + i * NW, NCHUNK - 1)

    def adj_cp(i, adj_v, sema):
        c = chunk_of(i)
        return pltpu.make_async_copy(
            adj_hbm.at[pl.ds(c * RPC, RPC)], adj_v, sema)

    def out_cp(i, acc_v, semo):
        c = chunk_of(i)
        return pltpu.make_async_copy(
            acc_v, out_hbm.at[pl.ds(c * PC, PC)], semo)

    def stage(adj_v, idx_v, acc_v, semr):
        def init_body(p, carry):
            for v in range(VPR):
                acc_v[p, pl.ds(v * L, L)] = bias_v[pl.ds(v * L, L)]
            return carry

        lax.fori_loop(0, PC, init_body, 0)

        for s in range(SPIRAL):
            av = adj_v[pl.ds(s * PC, PC)]
            idx_v[s, pl.ds(0, PC)] = av + s * N_PTS
            pltpu.async_copy(y_hbm.at[idx_v.at[s]], acc_v, semr, add=True)

    def drain(idx_v, acc_v, semr):
        def d_body(s, carry):
            pltpu.make_async_copy(y_hbm.at[idx_v.at[0]], acc_v, semr).wait()
            return carry

        lax.fori_loop(0, SPIRAL, d_body, 0)

    def elu_zero(i, acc_v):
        c = chunk_of(i)

        def e_body(p, carry):
            for v in range(VPR):
                z = acc_v[p, pl.ds(v * L, L)]
                acc_v[p, pl.ds(v * L, L)] = jnp.where(
                    z > 0.0, z, jnp.exp(jnp.minimum(z, 0.0)) - 1.0)
            return carry

        lax.fori_loop(0, PC, e_body, 0)

        @pl.when(c == NCHUNK - 1)
        def _():
            zero = jnp.zeros((L,), jnp.float32)
            for v in range(VPR):
                acc_v[PC - 1, pl.ds(v * L, L)] = zero

    def valid(i):
        return wid + i * NW < NCHUNK

    def step(i, cur, nxt):
        (c_adj, c_idx, c_acc, c_sema, c_semr, c_semo) = cur
        (n_adj, n_idx, n_acc, n_sema, n_semr, n_semo) = nxt

        @pl.when(i + 1 < NIT)
        def _():
            @pl.when((i >= 1) & valid(i - 1))
            def _():
                out_cp(i - 1, n_acc, n_semo).wait()

            adj_cp(i + 1, n_adj, n_sema).wait()
            stage(n_adj, n_idx, n_acc, n_semr)

        @pl.when(i + 2 < NIT)
        def _():
            adj_cp(i + 2, c_adj, c_sema).start()

        drain(c_idx, c_acc, c_semr)
        elu_zero(i, c_acc)

        @pl.when(valid(i))
        def _():
            out_cp(i, c_acc, c_semo).start()

    slot0 = (adj0, idx0, acc0, sema0, semr0, semo0)
    slot1 = (adj1, idx1, acc1, sema1, semr1, semo1)

    adj_cp(0, adj0, sema0).start()
    adj_cp(0, adj0, sema0).wait()
    stage(adj0, idx0, acc0, semr0)
    adj_cp(1, adj1, sema1).start()

    def pair_body(g, carry):
        step(2 * g, slot0, slot1)
        step(2 * g + 1, slot1, slot0)
        return carry

    lax.fori_loop(0, NIT // 2, pair_body, 0)

    @pl.when(valid(NIT - 2))
    def _():
        out_cp(NIT - 2, acc0 if (NIT - 2) % 2 == 0 else acc1,
               semo0 if (NIT - 2) % 2 == 0 else semo1).wait()

    @pl.when(valid(NIT - 1))
    def _():
        out_cp(NIT - 1, acc0 if (NIT - 1) % 2 == 0 else acc1,
               semo0 if (NIT - 1) % 2 == 0 else semo1).wait()


def kernel(x, spiral_adj, W, b):
    x2d = x.reshape(N_PTS, IN_C).astype(jnp.bfloat16)
    adj = (spiral_adj.reshape(NCHUNK, PC, SPIRAL).astype(jnp.int32)
           .transpose(0, 2, 1).reshape(N_PTS * SPIRAL))
    wmat3 = (W.reshape(OUT_C, SPIRAL, IN_C).transpose(1, 2, 0)
             .astype(jnp.bfloat16))
    y = _matmul(x2d, wmat3)
    yr = y.reshape(SPIRAL * N_PTS, OUT_C)
    out2d = _sc_gather(yr, adj, b)
    return out2d.reshape(1, N_PTS, OUT_C)
